# baseline (speedup 1.0000x reference)
import jax
import jax.numpy as jnp
from jax import lax
from jax.experimental import pallas as pl
from jax.experimental.pallas import tpu as pltpu
from jax.experimental.pallas import tpu_sc as plsc

HID = 256
N = 10000
NP = 10240
E = 160000
EP = 163840
LAYERS = 3
NUM_CLASS = 2

NS = 16
CHUNK = 128
NCHUNK = EP // NS // CHUNK
RPT = NP // NS
RCH = RPT // CHUNK
NB = 2
SCH = 16
NSTAGE = NCHUNK // SCH

_F32 = jnp.float32

_SC_PARAMS = pltpu.CompilerParams(use_tc_tiling_on_sc=True)


def _make_seg_sum():
    mesh = plsc.VectorSubcoreMesh(core_axis_name="c", subcore_axis_name="s")

    out_type = [jax.ShapeDtypeStruct((NP, 128), _F32)] * 2
    scratch = [
        pltpu.VMEM((NB, SCH, CHUNK), jnp.int32),
        pltpu.VMEM((NB, SCH, CHUNK), jnp.int32),
        pltpu.VMEM((NB, CHUNK, 128), _F32),
        pltpu.VMEM_SHARED((NP, 128), _F32),
        pltpu.SemaphoreType.DMA,
    ] + [pltpu.SemaphoreType.DMA] * (2 * NB)
    def body(y_lo, y_hi, src3, dst3, out_lo, out_hi,
             srcq, dstq, rows, acc, isem, g0, g1, s0, s1):
        sems = [g0, g1]
        ssems = [s0, s1]
        c = lax.axis_index("c")
        s = lax.axis_index("s")
        base = s * RPT

        zeros16 = jnp.zeros((16,), _F32)

        def _zero_row(r, _):
            def _zero_col(cc, _):
                rows[0, r, pl.ds(cc * 16, 16)] = zeros16
                return 0
            return lax.fori_loop(0, 128 // 16, _zero_col, 0)

        lax.fori_loop(0, CHUNK, _zero_row, 0)
        for j in range(RCH):
            pltpu.sync_copy(rows.at[0], acc.at[pl.ds(base + j * CHUNK, CHUNK)])

        def stage_copy(q, ib):
            r0 = q * SCH
            return [pltpu.async_copy(src3.at[s, pl.ds(r0, SCH)], srcq.at[ib],
                                     isem),
                    pltpu.async_copy(dst3.at[s, pl.ds(r0, SCH)], dstq.at[ib],
                                     isem)]

        for d in stage_copy(0, 0):
            d.wait()
        plsc.subcore_barrier()

        def scat_desc(ib, j, b):
            return pltpu.make_async_copy(rows.at[b], acc.at[dstq.at[ib, j]],
                                         ssems[b])

        def fire(ib, j, b, ws=True):
            if ws:
                scat_desc(ib, j, b).wait()

            @pl.when(c == 0)
            def _():
                pltpu.async_copy(y_lo.at[srcq.at[ib, j]], rows.at[b], sems[b])

            @pl.when(c == 1)
            def _():
                pltpu.async_copy(y_hi.at[srcq.at[ib, j]], rows.at[b], sems[b])

        def drain(ib, j, b):
            pltpu.make_async_copy(y_lo.at[srcq.at[ib, j]], rows.at[b],
                                  sems[b]).wait()
            pltpu.async_copy(rows.at[b], acc.at[dstq.at[ib, j]], ssems[b],
                             add=True)

        for b in range(NB):
            fire(0, b, b, ws=False)
        for q in range(NSTAGE):
            ib = q % 2
            nxt = stage_copy(q + 1, 1 - ib) if q + 1 < NSTAGE else []

            def steady(t, _):
                for b in range(NB):
                    j = t * NB + b
                    drain(ib, j, b)
                    fire(ib, j + NB, b)
                return 0

            lax.fori_loop(0, (SCH - NB) // NB, steady, 0)
            for d in nxt:
                d.wait()
            for b in range(NB):
                drain(ib, SCH - NB + b, b)
                if nxt:
                    fire(1 - ib, b, b)

        for b in range(NB):
            scat_desc((NSTAGE - 1) % 2, SCH - NB + b, b).wait()
        plsc.subcore_barrier()

        for j in range(RCH):
            r0 = base + j * CHUNK
            pltpu.sync_copy(acc.at[pl.ds(r0, CHUNK)], rows.at[0])

            @pl.when(c == 0)
            def _():
                pltpu.sync_copy(rows.at[0], out_lo.at[pl.ds(r0, CHUNK)])

            @pl.when(c == 1)
            def _():
                pltpu.sync_copy(rows.at[0], out_hi.at[pl.ds(r0, CHUNK)])

    return pl.kernel(body, out_type=out_type, mesh=mesh, scratch_types=scratch,
                     compiler_params=_SC_PARAMS,
                     cost_estimate=pl.CostEstimate(
                         flops=2 * EP * 128, transcendentals=0,
                         bytes_accessed=4 * EP * 128 * 4))


def _make_deg():
    mesh = plsc.VectorSubcoreMesh(core_axis_name="c", subcore_axis_name="s")
    scratch = [
        pltpu.VMEM((NCHUNK, CHUNK), jnp.int32),
        pltpu.VMEM((CHUNK, 16), _F32),
        pltpu.VMEM((CHUNK, 16), _F32),
        pltpu.VMEM_SHARED((NP, 16), _F32),
    ]

    def body(dst3, deg_out, dstb, ones_v, zb, deg_acc):
        c = lax.axis_index("c")
        s = lax.axis_index("s")
        base = s * RPT
        zeros16 = jnp.zeros((16,), _F32)
        ones16 = jnp.ones((16,), _F32)

        def _fill(r, _):
            ones_v[r, :] = ones16
            zb[r, :] = zeros16
            return 0

        lax.fori_loop(0, CHUNK, _fill, 0)
        pltpu.sync_copy(dst3.at[s], dstb)
        for j in range(RCH):
            pltpu.sync_copy(zb, deg_acc.at[pl.ds(base + j * CHUNK, CHUNK)])
        plsc.subcore_barrier()

        @pl.when(c == 0)
        def _():
            def step(k, _):
                pltpu.sync_copy(ones_v, deg_acc.at[dstb.at[k]], add=True)
                return 0
            lax.fori_loop(0, NCHUNK, step, 0)

        plsc.subcore_barrier()

        @pl.when(c == 0)
        def _():
            for j in range(RCH):
                r0 = base + j * CHUNK
                pltpu.sync_copy(deg_acc.at[pl.ds(r0, CHUNK)], zb)
                pltpu.sync_copy(zb, deg_out.at[pl.ds(r0, CHUNK)])

    return pl.kernel(body, out_type=jax.ShapeDtypeStruct((NP, 16), _F32),
                     mesh=mesh, scratch_types=scratch,
                     compiler_params=_SC_PARAMS,
                     cost_estimate=pl.CostEstimate(
                         flops=EP * 16, transcendentals=0,
                         bytes_accessed=2 * EP * 16 * 4))


_seg_sum = _make_seg_sum()
_deg = _make_deg()


BR = 1000
GRID = N // BR


def _ln_val(t, g, b):
    mu = jnp.mean(t, axis=-1, keepdims=True)
    d = t - mu
    var = jnp.mean(d * d, axis=-1, keepdims=True)
    return d * lax.rsqrt(var + 1e-5) * g + b


def _leaky_val(t):
    return jnp.where(t >= 0, t, 0.01 * t)


def _row_spec(width):
    return pl.BlockSpec((BR, width), lambda i: (i, 0))


def _full_spec(shape):
    return pl.BlockSpec(shape, lambda i: (0,) * len(shape))


def _halves(y, os):
    os[0][...] = y[:, :128]
    os[1][...] = y[:, 128:]


def _mm_ln_split(x, W, b, g, bt, hW):
    K = x.shape[1]

    def body(x_ref, w_ref, b_ref, g_ref, bt_ref, hw_ref, o_ref, *oh):
        t = jnp.dot(x_ref[...], w_ref[...], preferred_element_type=_F32)
        x0 = _ln_val(t + b_ref[...], g_ref[...], bt_ref[...])
        o_ref[...] = x0
        _halves(jnp.dot(x0, hw_ref[...], preferred_element_type=_F32), oh)

    return pl.pallas_call(
        body,
        grid=(GRID,),
        in_specs=[_row_spec(K), _full_spec((K, HID)), _full_spec((1, HID)),
                  _full_spec((1, HID)), _full_spec((1, HID)),
                  _full_spec((HID, HID))],
        out_specs=[_row_spec(HID)] + [_row_spec(128)] * 2,
        out_shape=[jax.ShapeDtypeStruct((N, HID), _F32)]
                  + [jax.ShapeDtypeStruct((N, 128), _F32)] * 2,
    )(x, W, b, g, bt, hW)


def _post_val(s_vals, deg_ref, hb_ref, w_refs, eb_ref, g_ref, bt_ref):
    hb = hb_ref[...]
    t = jnp.dot(s_vals[0], w_refs[0][...], preferred_element_type=_F32)
    bias = jnp.dot(hb[:, :128], w_refs[0][...], preferred_element_type=_F32)
    t += jnp.dot(s_vals[1], w_refs[1][...], preferred_element_type=_F32)
    bias += jnp.dot(hb[:, 128:], w_refs[1][...], preferred_element_type=_F32)
    inv = 1.0 / jnp.maximum(deg_ref[:, 0:1], 1.0)
    t = t * inv + bias + eb_ref[...]
    return _leaky_val(_ln_val(t, g_ref[...], bt_ref[...]))


_POST_SPECS = ([_row_spec(128)] * 2 + [_row_spec(16), _full_spec((1, HID))]
               + [_full_spec((128, HID))] * 2 + [_full_spec((1, HID))] * 3)


def _mm_parts(parts, Ws):
    np_ = len(parts)

    def body(*refs):
        part_refs, w_refs, o_ref = refs[:np_], refs[np_:2 * np_], refs[-1]
        t = jnp.dot(part_refs[0][...], w_refs[0][...],
                    preferred_element_type=_F32)
        for j in range(1, np_):
            t += jnp.dot(part_refs[j][...], w_refs[j][...],
                         preferred_element_type=_F32)
        o_ref[...] = t

    return pl.pallas_call(
        body,
        grid=(GRID,),
        in_specs=[_row_spec(HID)] * np_ + [_full_spec((HID, HID))] * np_,
        out_specs=_row_spec(HID),
        out_shape=jax.ShapeDtypeStruct((N, HID), _F32),
    )(*parts, *Ws)


def _post_mm(shs, deg, hb, eW, eb, eg, ebt, ypart, hW_last):
    def body(*refs):
        i = 0
        s_refs = refs[i:i + 2]; i += 2
        deg_ref, hb_ref = refs[i:i + 2]; i += 2
        ew_refs = refs[i:i + 2]; i += 2
        eb_ref, g_ref, bt_ref = refs[i:i + 3]; i += 3
        yp_ref, hw_ref = refs[i:i + 2]; i += 2
        h_ref = refs[i]; i += 1
        oh = refs[i:]

        h = _post_val([r[...] for r in s_refs], deg_ref, hb_ref, ew_refs,
                      eb_ref, g_ref, bt_ref)
        h_ref[...] = h
        y = yp_ref[...] + jnp.dot(h, hw_ref[...], preferred_element_type=_F32)
        _halves(y, oh)

    return pl.pallas_call(
        body,
        grid=(GRID,),
        in_specs=_POST_SPECS + [_row_spec(HID), _full_spec((HID, HID))],
        out_specs=[_row_spec(HID)] + [_row_spec(128)] * 2,
        out_shape=[jax.ShapeDtypeStruct((N, HID), _F32)]
                  + [jax.ShapeDtypeStruct((N, 128), _F32)] * 2,
    )(*shs, deg, hb, eW[:128], eW[128:], eb, eg, ebt, ypart, hW_last)


def _post_head(shs, deg, hb, eW, eb, eg, ebt, tpart, W3_last,
               b3, g3, bt3, fcW_pad, fcb_pad):
    def body(*refs):
        i = 0
        s_refs = refs[i:i + 2]; i += 2
        deg_ref, hb_ref = refs[i:i + 2]; i += 2
        ew_refs = refs[i:i + 2]; i += 2
        eb_ref, g_ref, bt_ref = refs[i:i + 3]; i += 3
        (tp_ref, w3_ref, b3_ref, g3_ref, bt3_ref,
         fw_ref, fb_ref, o_ref) = refs[i:]

        h = _post_val([r[...] for r in s_refs], deg_ref, hb_ref, ew_refs,
                      eb_ref, g_ref, bt_ref)
        t = tp_ref[...] + jnp.dot(h, w3_ref[...], preferred_element_type=_F32)
        z = _leaky_val(_ln_val(t + b3_ref[...], g3_ref[...], bt3_ref[...]))
        o_ref[...] = jnp.dot(z, fw_ref[...], preferred_element_type=_F32) \
            + fb_ref[...]

    return pl.pallas_call(
        body,
        grid=(GRID,),
        in_specs=_POST_SPECS
                 + [_row_spec(HID), _full_spec((HID, HID))]
                 + [_full_spec((1, HID))] * 3
                 + [_full_spec((HID, 128)), _full_spec((1, 128))],
        out_specs=_row_spec(128),
        out_shape=jax.ShapeDtypeStruct((N, 128), _F32),
    )(*shs, deg, hb, eW[:128], eW[128:], eb, eg, ebt, tpart, W3_last,
      b3, g3, bt3, fcW_pad, fcb_pad)


def kernel(x_all, edge_index, p, fc1_W, fc1_b, ln1_g, ln1_b,
           hW0, hb0, eW0, eb0, eg0, ebt0,
           hW1, hb1, eW1, eb1, eg1, ebt1,
           hW2, hb2, eW2, eb2, eg2, ebt2,
           fc3_W, fc3_b, ln3_g, ln3_b, fc_W, fc_b):
    del p
    hWs = [hW0, hW1, hW2]
    hbs = [hb0, hb1, hb2]
    eWs = [eW0, eW1, eW2]
    ebs = [eb0, eb1, eb2]
    egs = [eg0, eg1, eg2]
    ebts = [ebt0, ebt1, ebt2]

    row = lambda v: v.reshape(1, -1)

    npad = EP - E
    ar = jnp.arange(npad, dtype=jnp.int32)
    src = jnp.concatenate([edge_index[0], (ar * 37) % N]).reshape(NS, NCHUNK, CHUNK)
    dst = jnp.concatenate([edge_index[1], N + (ar % 16)]).reshape(NS, NCHUNK, CHUNK)

    deg = _deg(dst)
    x0, *yhs = _mm_ln_split(x_all, fc1_W, row(fc1_b), row(ln1_g), row(ln1_b),
                            hW0)

    def wsplit(W, n):
        return [lax.slice(W, (j * HID, 0), ((j + 1) * HID, HID))
                for j in range(n)]

    parts = [x0]
    for i in range(LAYERS - 1):
        shs = _seg_sum(*yhs, src, dst)
        hWn = wsplit(hWs[i + 1], i + 2)
        ypart = _mm_parts(parts, hWn[:i + 1])
        h, *yhs = _post_mm(shs, deg, row(hbs[i]), eWs[i], row(ebs[i]),
                           row(egs[i]), row(ebts[i]), ypart, hWn[i + 1])
        parts.append(h)

    shs = _seg_sum(*yhs, src, dst)
    W3s = wsplit(fc3_W, LAYERS + 1)
    tpart = _mm_parts(parts, W3s[:LAYERS])
    fcW_pad = jnp.pad(fc_W, ((0, 0), (0, 128 - NUM_CLASS)))
    fcb_pad = jnp.pad(fc_b, ((0, 128 - NUM_CLASS),))
    out = _post_head(shs, deg, row(hbs[2]), eWs[2], row(ebs[2]),
                     row(egs[2]), row(ebts[2]), tpart, W3s[LAYERS],
                     row(fc3_b), row(ln3_g), row(ln3_b),
                     fcW_pad, row(fcb_pad))
    return out[:, :NUM_CLASS]


# speedup vs baseline: 1.0003x; 1.0003x over previous
import jax
import jax.numpy as jnp
from jax import lax
from jax.experimental import pallas as pl
from jax.experimental.pallas import tpu as pltpu
from jax.experimental.pallas import tpu_sc as plsc

HID = 256
N = 10000
NP = 10240
E = 160000
EP = 163840
LAYERS = 3
NUM_CLASS = 2

NS = 16
CHUNK = 128
NCHUNK = EP // NS // CHUNK
RPT = NP // NS
RCH = RPT // CHUNK
NB = 2
SCH = 16
NSTAGE = NCHUNK // SCH

_F32 = jnp.float32

_SC_PARAMS = pltpu.CompilerParams(use_tc_tiling_on_sc=True)


def _make_seg_sum():
    mesh = plsc.VectorSubcoreMesh(core_axis_name="c", subcore_axis_name="s")

    out_type = [jax.ShapeDtypeStruct((NP, 128), _F32)] * 2
    scratch = [
        pltpu.VMEM((NB, SCH, CHUNK), jnp.int32),
        pltpu.VMEM((NB, SCH, CHUNK), jnp.int32),
        pltpu.VMEM((NB, CHUNK, 128), _F32),
        pltpu.VMEM_SHARED((NP, 128), _F32),
        pltpu.SemaphoreType.DMA,
    ] + [pltpu.SemaphoreType.DMA] * (2 * NB)
    def body(y_lo, y_hi, src3, dst3, out_lo, out_hi,
             srcq, dstq, rows, acc, isem, g0, g1, s0, s1):
        sems = [g0, g1]
        ssems = [s0, s1]
        c = lax.axis_index("c")
        s = lax.axis_index("s")
        base = s * RPT

        zeros16 = jnp.zeros((16,), _F32)

        def _zero_row(r, _):
            def _zero_col(cc, _):
                rows[0, r, pl.ds(cc * 16, 16)] = zeros16
                return 0
            return lax.fori_loop(0, 128 // 16, _zero_col, 0)

        lax.fori_loop(0, CHUNK, _zero_row, 0)
        for j in range(RCH):
            pltpu.sync_copy(rows.at[0], acc.at[pl.ds(base + j * CHUNK, CHUNK)])

        def stage_copy(q, ib):
            r0 = q * SCH
            return [pltpu.async_copy(src3.at[s, pl.ds(r0, SCH)], srcq.at[ib],
                                     isem),
                    pltpu.async_copy(dst3.at[s, pl.ds(r0, SCH)], dstq.at[ib],
                                     isem)]

        for d in stage_copy(0, 0):
            d.wait()
        plsc.subcore_barrier()

        def scat_desc(ib, j, b):
            return pltpu.make_async_copy(rows.at[b], acc.at[dstq.at[ib, j]],
                                         ssems[b])

        def fire(ib, j, b, ws=True):
            if ws:
                scat_desc(ib, j, b).wait()

            @pl.when(c == 0)
            def _():
                for hh in range(2):
                    pltpu.async_copy(y_lo.at[srcq.at[ib, j, pl.ds(hh * 64, 64)]],
                                     rows.at[b, pl.ds(hh * 64, 64)], sems[b])

            @pl.when(c == 1)
            def _():
                for hh in range(2):
                    pltpu.async_copy(y_hi.at[srcq.at[ib, j, pl.ds(hh * 64, 64)]],
                                     rows.at[b, pl.ds(hh * 64, 64)], sems[b])

        def drain(ib, j, b):
            for hh in range(2):
                pltpu.make_async_copy(y_lo.at[srcq.at[ib, j, pl.ds(hh * 64, 64)]],
                                      rows.at[b, pl.ds(hh * 64, 64)],
                                      sems[b]).wait()
            pltpu.async_copy(rows.at[b], acc.at[dstq.at[ib, j]], ssems[b],
                             add=True)

        for b in range(NB):
            fire(0, b, b, ws=False)
        for q in range(NSTAGE):
            ib = q % 2
            nxt = stage_copy(q + 1, 1 - ib) if q + 1 < NSTAGE else []

            def steady(t, _):
                for b in range(NB):
                    j = t * NB + b
                    drain(ib, j, b)
                    fire(ib, j + NB, b)
                return 0

            lax.fori_loop(0, (SCH - NB) // NB, steady, 0)
            for d in nxt:
                d.wait()
            for b in range(NB):
                drain(ib, SCH - NB + b, b)
                if nxt:
                    fire(1 - ib, b, b)

        for b in range(NB):
            scat_desc((NSTAGE - 1) % 2, SCH - NB + b, b).wait()
        plsc.subcore_barrier()

        for j in range(RCH):
            r0 = base + j * CHUNK
            pltpu.sync_copy(acc.at[pl.ds(r0, CHUNK)], rows.at[0])

            @pl.when(c == 0)
            def _():
                pltpu.sync_copy(rows.at[0], out_lo.at[pl.ds(r0, CHUNK)])

            @pl.when(c == 1)
            def _():
                pltpu.sync_copy(rows.at[0], out_hi.at[pl.ds(r0, CHUNK)])

    return pl.kernel(body, out_type=out_type, mesh=mesh, scratch_types=scratch,
                     compiler_params=_SC_PARAMS,
                     cost_estimate=pl.CostEstimate(
                         flops=2 * EP * 128, transcendentals=0,
                         bytes_accessed=4 * EP * 128 * 4))


def _make_deg():
    mesh = plsc.VectorSubcoreMesh(core_axis_name="c", subcore_axis_name="s")
    scratch = [
        pltpu.VMEM((NCHUNK, CHUNK), jnp.int32),
        pltpu.VMEM((CHUNK, 16), _F32),
        pltpu.VMEM((CHUNK, 16), _F32),
        pltpu.VMEM_SHARED((NP, 16), _F32),
    ]

    def body(dst3, deg_out, dstb, ones_v, zb, deg_acc):
        c = lax.axis_index("c")
        s = lax.axis_index("s")
        base = s * RPT
        zeros16 = jnp.zeros((16,), _F32)
        ones16 = jnp.ones((16,), _F32)

        def _fill(r, _):
            ones_v[r, :] = ones16
            zb[r, :] = zeros16
            return 0

        lax.fori_loop(0, CHUNK, _fill, 0)
        pltpu.sync_copy(dst3.at[s], dstb)
        for j in range(RCH):
            pltpu.sync_copy(zb, deg_acc.at[pl.ds(base + j * CHUNK, CHUNK)])
        plsc.subcore_barrier()

        @pl.when(c == 0)
        def _():
            def step(k, _):
                pltpu.sync_copy(ones_v, deg_acc.at[dstb.at[k]], add=True)
                return 0
            lax.fori_loop(0, NCHUNK, step, 0)

        plsc.subcore_barrier()

        @pl.when(c == 0)
        def _():
            for j in range(RCH):
                r0 = base + j * CHUNK
                pltpu.sync_copy(deg_acc.at[pl.ds(r0, CHUNK)], zb)
                pltpu.sync_copy(zb, deg_out.at[pl.ds(r0, CHUNK)])

    return pl.kernel(body, out_type=jax.ShapeDtypeStruct((NP, 16), _F32),
                     mesh=mesh, scratch_types=scratch,
                     compiler_params=_SC_PARAMS,
                     cost_estimate=pl.CostEstimate(
                         flops=EP * 16, transcendentals=0,
                         bytes_accessed=2 * EP * 16 * 4))


_seg_sum = _make_seg_sum()
_deg = _make_deg()


BR = 1000
GRID = N // BR


def _ln_val(t, g, b):
    mu = jnp.mean(t, axis=-1, keepdims=True)
    d = t - mu
    var = jnp.mean(d * d, axis=-1, keepdims=True)
    return d * lax.rsqrt(var + 1e-5) * g + b


def _leaky_val(t):
    return jnp.where(t >= 0, t, 0.01 * t)


def _row_spec(width):
    return pl.BlockSpec((BR, width), lambda i: (i, 0))


def _full_spec(shape):
    return pl.BlockSpec(shape, lambda i: (0,) * len(shape))


def _halves(y, os):
    os[0][...] = y[:, :128]
    os[1][...] = y[:, 128:]


def _mm_ln_split(x, W, b, g, bt, hW):
    K = x.shape[1]

    def body(x_ref, w_ref, b_ref, g_ref, bt_ref, hw_ref, o_ref, *oh):
        t = jnp.dot(x_ref[...], w_ref[...], preferred_element_type=_F32)
        x0 = _ln_val(t + b_ref[...], g_ref[...], bt_ref[...])
        o_ref[...] = x0
        _halves(jnp.dot(x0, hw_ref[...], preferred_element_type=_F32), oh)

    return pl.pallas_call(
        body,
        grid=(GRID,),
        in_specs=[_row_spec(K), _full_spec((K, HID)), _full_spec((1, HID)),
                  _full_spec((1, HID)), _full_spec((1, HID)),
                  _full_spec((HID, HID))],
        out_specs=[_row_spec(HID)] + [_row_spec(128)] * 2,
        out_shape=[jax.ShapeDtypeStruct((N, HID), _F32)]
                  + [jax.ShapeDtypeStruct((N, 128), _F32)] * 2,
    )(x, W, b, g, bt, hW)


def _post_val(s_vals, deg_ref, hb_ref, w_refs, eb_ref, g_ref, bt_ref):
    hb = hb_ref[...]
    t = jnp.dot(s_vals[0], w_refs[0][...], preferred_element_type=_F32)
    bias = jnp.dot(hb[:, :128], w_refs[0][...], preferred_element_type=_F32)
    t += jnp.dot(s_vals[1], w_refs[1][...], preferred_element_type=_F32)
    bias += jnp.dot(hb[:, 128:], w_refs[1][...], preferred_element_type=_F32)
    inv = 1.0 / jnp.maximum(deg_ref[:, 0:1], 1.0)
    t = t * inv + bias + eb_ref[...]
    return _leaky_val(_ln_val(t, g_ref[...], bt_ref[...]))


_POST_SPECS = ([_row_spec(128)] * 2 + [_row_spec(16), _full_spec((1, HID))]
               + [_full_spec((128, HID))] * 2 + [_full_spec((1, HID))] * 3)


def _mm_parts(parts, Ws):
    np_ = len(parts)

    def body(*refs):
        part_refs, w_refs, o_ref = refs[:np_], refs[np_:2 * np_], refs[-1]
        t = jnp.dot(part_refs[0][...], w_refs[0][...],
                    preferred_element_type=_F32)
        for j in range(1, np_):
            t += jnp.dot(part_refs[j][...], w_refs[j][...],
                         preferred_element_type=_F32)
        o_ref[...] = t

    return pl.pallas_call(
        body,
        grid=(GRID,),
        in_specs=[_row_spec(HID)] * np_ + [_full_spec((HID, HID))] * np_,
        out_specs=_row_spec(HID),
        out_shape=jax.ShapeDtypeStruct((N, HID), _F32),
    )(*parts, *Ws)


def _post_mm(shs, deg, hb, eW, eb, eg, ebt, ypart, hW_last):
    def body(*refs):
        i = 0
        s_refs = refs[i:i + 2]; i += 2
        deg_ref, hb_ref = refs[i:i + 2]; i += 2
        ew_refs = refs[i:i + 2]; i += 2
        eb_ref, g_ref, bt_ref = refs[i:i + 3]; i += 3
        yp_ref, hw_ref = refs[i:i + 2]; i += 2
        h_ref = refs[i]; i += 1
        oh = refs[i:]

        h = _post_val([r[...] for r in s_refs], deg_ref, hb_ref, ew_refs,
                      eb_ref, g_ref, bt_ref)
        h_ref[...] = h
        y = yp_ref[...] + jnp.dot(h, hw_ref[...], preferred_element_type=_F32)
        _halves(y, oh)

    return pl.pallas_call(
        body,
        grid=(GRID,),
        in_specs=_POST_SPECS + [_row_spec(HID), _full_spec((HID, HID))],
        out_specs=[_row_spec(HID)] + [_row_spec(128)] * 2,
        out_shape=[jax.ShapeDtypeStruct((N, HID), _F32)]
                  + [jax.ShapeDtypeStruct((N, 128), _F32)] * 2,
    )(*shs, deg, hb, eW[:128], eW[128:], eb, eg, ebt, ypart, hW_last)


def _post_head(shs, deg, hb, eW, eb, eg, ebt, tpart, W3_last,
               b3, g3, bt3, fcW_pad, fcb_pad):
    def body(*refs):
        i = 0
        s_refs = refs[i:i + 2]; i += 2
        deg_ref, hb_ref = refs[i:i + 2]; i += 2
        ew_refs = refs[i:i + 2]; i += 2
        eb_ref, g_ref, bt_ref = refs[i:i + 3]; i += 3
        (tp_ref, w3_ref, b3_ref, g3_ref, bt3_ref,
         fw_ref, fb_ref, o_ref) = refs[i:]

        h = _post_val([r[...] for r in s_refs], deg_ref, hb_ref, ew_refs,
                      eb_ref, g_ref, bt_ref)
        t = tp_ref[...] + jnp.dot(h, w3_ref[...], preferred_element_type=_F32)
        z = _leaky_val(_ln_val(t + b3_ref[...], g3_ref[...], bt3_ref[...]))
        o_ref[...] = jnp.dot(z, fw_ref[...], preferred_element_type=_F32) \
            + fb_ref[...]

    return pl.pallas_call(
        body,
        grid=(GRID,),
        in_specs=_POST_SPECS
                 + [_row_spec(HID), _full_spec((HID, HID))]
                 + [_full_spec((1, HID))] * 3
                 + [_full_spec((HID, 128)), _full_spec((1, 128))],
        out_specs=_row_spec(128),
        out_shape=jax.ShapeDtypeStruct((N, 128), _F32),
    )(*shs, deg, hb, eW[:128], eW[128:], eb, eg, ebt, tpart, W3_last,
      b3, g3, bt3, fcW_pad, fcb_pad)


def kernel(x_all, edge_index, p, fc1_W, fc1_b, ln1_g, ln1_b,
           hW0, hb0, eW0, eb0, eg0, ebt0,
           hW1, hb1, eW1, eb1, eg1, ebt1,
           hW2, hb2, eW2, eb2, eg2, ebt2,
           fc3_W, fc3_b, ln3_g, ln3_b, fc_W, fc_b):
    del p
    hWs = [hW0, hW1, hW2]
    hbs = [hb0, hb1, hb2]
    eWs = [eW0, eW1, eW2]
    ebs = [eb0, eb1, eb2]
    egs = [eg0, eg1, eg2]
    ebts = [ebt0, ebt1, ebt2]

    row = lambda v: v.reshape(1, -1)

    npad = EP - E
    ar = jnp.arange(npad, dtype=jnp.int32)
    src = jnp.concatenate([edge_index[0], (ar * 37) % N]).reshape(NS, NCHUNK, CHUNK)
    dst = jnp.concatenate([edge_index[1], N + (ar % 16)]).reshape(NS, NCHUNK, CHUNK)

    deg = _deg(dst)
    x0, *yhs = _mm_ln_split(x_all, fc1_W, row(fc1_b), row(ln1_g), row(ln1_b),
                            hW0)

    def wsplit(W, n):
        return [lax.slice(W, (j * HID, 0), ((j + 1) * HID, HID))
                for j in range(n)]

    parts = [x0]
    for i in range(LAYERS - 1):
        shs = _seg_sum(*yhs, src, dst)
        hWn = wsplit(hWs[i + 1], i + 2)
        ypart = _mm_parts(parts, hWn[:i + 1])
        h, *yhs = _post_mm(shs, deg, row(hbs[i]), eWs[i], row(ebs[i]),
                           row(egs[i]), row(ebts[i]), ypart, hWn[i + 1])
        parts.append(h)

    shs = _seg_sum(*yhs, src, dst)
    W3s = wsplit(fc3_W, LAYERS + 1)
    tpart = _mm_parts(parts, W3s[:LAYERS])
    fcW_pad = jnp.pad(fc_W, ((0, 0), (0, 128 - NUM_CLASS)))
    fcb_pad = jnp.pad(fc_b, ((0, 128 - NUM_CLASS),))
    out = _post_head(shs, deg, row(hbs[2]), eWs[2], row(ebs[2]),
                     row(egs[2]), row(ebts[2]), tpart, W3s[LAYERS],
                     row(fc3_b), row(ln3_g), row(ln3_b),
                     fcW_pad, row(fcb_pad))
    return out[:, :NUM_CLASS]


# speedup vs baseline: 1.0098x; 1.0095x over previous
import jax
import jax.numpy as jnp
from jax import lax
from jax.experimental import pallas as pl
from jax.experimental.pallas import tpu as pltpu
from jax.experimental.pallas import tpu_sc as plsc

HID = 256
N = 10000
NP = 10240
E = 160000
EP = 163840
LAYERS = 3
NUM_CLASS = 2

NS = 16
CHUNK = 128
NCHUNK = EP // NS // CHUNK
RPT = NP // NS
RCH = RPT // CHUNK
NB = 2
SCH = 16
NSTAGE = NCHUNK // SCH

_F32 = jnp.float32

_SC_PARAMS = pltpu.CompilerParams(use_tc_tiling_on_sc=True)


def _make_seg_sum():
    mesh = plsc.VectorSubcoreMesh(core_axis_name="c", subcore_axis_name="s")

    out_type = [jax.ShapeDtypeStruct((NP, 128), _F32)] * 2
    scratch = [
        pltpu.VMEM((NB, SCH, CHUNK), jnp.int32),
        pltpu.VMEM((NB, SCH, CHUNK), jnp.int32),
        pltpu.VMEM((NB, CHUNK, 128), _F32),
        pltpu.VMEM_SHARED((NP, 128), _F32),
        pltpu.SemaphoreType.DMA,
    ] + [pltpu.SemaphoreType.DMA] * (2 * NB)
    def body(y_lo, y_hi, src3, dst3, out_lo, out_hi,
             srcq, dstq, rows, acc, isem, g0, g1, s0, s1):
        sems = [g0, g1]
        ssems = [s0, s1]
        c = lax.axis_index("c")
        s = lax.axis_index("s")
        base = s * RPT

        zeros16 = jnp.zeros((16,), _F32)

        def _zero_row(r, _):
            def _zero_col(cc, _):
                rows[0, r, pl.ds(cc * 16, 16)] = zeros16
                return 0
            return lax.fori_loop(0, 128 // 16, _zero_col, 0)

        lax.fori_loop(0, CHUNK, _zero_row, 0)
        for j in range(RCH):
            pltpu.sync_copy(rows.at[0], acc.at[pl.ds(base + j * CHUNK, CHUNK)])

        def stage_copy(q, ib):
            r0 = q * SCH
            return [pltpu.async_copy(src3.at[s, pl.ds(r0, SCH)], srcq.at[ib],
                                     isem),
                    pltpu.async_copy(dst3.at[s, pl.ds(r0, SCH)], dstq.at[ib],
                                     isem)]

        for d in stage_copy(0, 0):
            d.wait()
        plsc.subcore_barrier()

        def scat_desc(ib, j, b):
            return pltpu.make_async_copy(rows.at[b], acc.at[dstq.at[ib, j]],
                                         ssems[b])

        def fire(ib, j, b, ws=True):
            if ws:
                scat_desc(ib, j, b).wait()

            @pl.when(c == 0)
            def _():
                pltpu.async_copy(y_lo.at[srcq.at[ib, j]], rows.at[b], sems[b])

            @pl.when(c == 1)
            def _():
                pltpu.async_copy(y_hi.at[srcq.at[ib, j]], rows.at[b], sems[b])

        def drain(ib, j, b):
            pltpu.make_async_copy(y_lo.at[srcq.at[ib, j]], rows.at[b],
                                  sems[b]).wait()
            pltpu.async_copy(rows.at[b], acc.at[dstq.at[ib, j]], ssems[b],
                             add=True)

        for b in range(NB):
            fire(0, b, b, ws=False)
        for q in range(NSTAGE):
            ib = q % 2
            nxt = stage_copy(q + 1, 1 - ib) if q + 1 < NSTAGE else []

            def steady(t, _):
                for b in range(NB):
                    j = t * NB + b
                    drain(ib, j, b)
                    fire(ib, j + NB, b)
                return 0

            lax.fori_loop(0, (SCH - NB) // NB, steady, 0)
            for d in nxt:
                d.wait()
            for b in range(NB):
                drain(ib, SCH - NB + b, b)
                if nxt:
                    fire(1 - ib, b, b)

        for b in range(NB):
            scat_desc((NSTAGE - 1) % 2, SCH - NB + b, b).wait()
        plsc.subcore_barrier()

        for j in range(RCH):
            r0 = base + j * CHUNK
            pltpu.sync_copy(acc.at[pl.ds(r0, CHUNK)], rows.at[0])

            @pl.when(c == 0)
            def _():
                pltpu.sync_copy(rows.at[0], out_lo.at[pl.ds(r0, CHUNK)])

            @pl.when(c == 1)
            def _():
                pltpu.sync_copy(rows.at[0], out_hi.at[pl.ds(r0, CHUNK)])

    return pl.kernel(body, out_type=out_type, mesh=mesh, scratch_types=scratch,
                     compiler_params=_SC_PARAMS,
                     cost_estimate=pl.CostEstimate(
                         flops=2 * EP * 128, transcendentals=0,
                         bytes_accessed=4 * EP * 128 * 4))


def _make_deg():
    mesh = plsc.VectorSubcoreMesh(core_axis_name="c", subcore_axis_name="s")
    scratch = [
        pltpu.VMEM((NCHUNK, CHUNK), jnp.int32),
        pltpu.VMEM((CHUNK, 16), _F32),
        pltpu.VMEM((CHUNK, 16), _F32),
        pltpu.VMEM_SHARED((NP, 16), _F32),
    ]

    def body(dst3, deg_out, dstb, ones_v, zb, deg_acc):
        c = lax.axis_index("c")
        s = lax.axis_index("s")
        base = s * RPT
        zeros16 = jnp.zeros((16,), _F32)
        ones16 = jnp.ones((16,), _F32)

        def _fill(r, _):
            ones_v[r, :] = ones16
            zb[r, :] = zeros16
            return 0

        lax.fori_loop(0, CHUNK, _fill, 0)
        pltpu.sync_copy(dst3.at[s], dstb)
        for j in range(RCH):
            pltpu.sync_copy(zb, deg_acc.at[pl.ds(base + j * CHUNK, CHUNK)])
        plsc.subcore_barrier()

        @pl.when(c == 0)
        def _():
            def step(k, _):
                pltpu.sync_copy(ones_v, deg_acc.at[dstb.at[k]], add=True)
                return 0
            lax.fori_loop(0, NCHUNK, step, 0)

        plsc.subcore_barrier()

        @pl.when(c == 0)
        def _():
            for j in range(RCH):
                r0 = base + j * CHUNK
                pltpu.sync_copy(deg_acc.at[pl.ds(r0, CHUNK)], zb)
                pltpu.sync_copy(zb, deg_out.at[pl.ds(r0, CHUNK)])

    return pl.kernel(body, out_type=jax.ShapeDtypeStruct((NP, 16), _F32),
                     mesh=mesh, scratch_types=scratch,
                     compiler_params=_SC_PARAMS,
                     cost_estimate=pl.CostEstimate(
                         flops=EP * 16, transcendentals=0,
                         bytes_accessed=2 * EP * 16 * 4))


_seg_sum = _make_seg_sum()
_deg = _make_deg()


BR = 2000
GRID = N // BR


def _ln_val(t, g, b):
    mu = jnp.mean(t, axis=-1, keepdims=True)
    d = t - mu
    var = jnp.mean(d * d, axis=-1, keepdims=True)
    return d * lax.rsqrt(var + 1e-5) * g + b


def _leaky_val(t):
    return jnp.where(t >= 0, t, 0.01 * t)


def _row_spec(width):
    return pl.BlockSpec((BR, width), lambda i: (i, 0))


def _full_spec(shape):
    return pl.BlockSpec(shape, lambda i: (0,) * len(shape))


def _halves(y, os):
    os[0][...] = y[:, :128]
    os[1][...] = y[:, 128:]


def _mm_ln_split(x, W, b, g, bt, hW):
    K = x.shape[1]

    def body(x_ref, w_ref, b_ref, g_ref, bt_ref, hw_ref, o_ref, *oh):
        t = jnp.dot(x_ref[...], w_ref[...], preferred_element_type=_F32)
        x0 = _ln_val(t + b_ref[...], g_ref[...], bt_ref[...])
        o_ref[...] = x0
        _halves(jnp.dot(x0, hw_ref[...], preferred_element_type=_F32), oh)

    return pl.pallas_call(
        body,
        grid=(GRID,),
        in_specs=[_row_spec(K), _full_spec((K, HID)), _full_spec((1, HID)),
                  _full_spec((1, HID)), _full_spec((1, HID)),
                  _full_spec((HID, HID))],
        out_specs=[_row_spec(HID)] + [_row_spec(128)] * 2,
        out_shape=[jax.ShapeDtypeStruct((N, HID), _F32)]
                  + [jax.ShapeDtypeStruct((N, 128), _F32)] * 2,
    )(x, W, b, g, bt, hW)


def _post_val(s_vals, deg_ref, hb_ref, w_refs, eb_ref, g_ref, bt_ref):
    hb = hb_ref[...]
    t = jnp.dot(s_vals[0], w_refs[0][...], preferred_element_type=_F32)
    bias = jnp.dot(hb[:, :128], w_refs[0][...], preferred_element_type=_F32)
    t += jnp.dot(s_vals[1], w_refs[1][...], preferred_element_type=_F32)
    bias += jnp.dot(hb[:, 128:], w_refs[1][...], preferred_element_type=_F32)
    inv = 1.0 / jnp.maximum(deg_ref[:, 0:1], 1.0)
    t = t * inv + bias + eb_ref[...]
    return _leaky_val(_ln_val(t, g_ref[...], bt_ref[...]))


_POST_SPECS = ([_row_spec(128)] * 2 + [_row_spec(16), _full_spec((1, HID))]
               + [_full_spec((128, HID))] * 2 + [_full_spec((1, HID))] * 3)


def _mm_parts(parts, Ws):
    np_ = len(parts)

    def body(*refs):
        part_refs, w_refs, o_ref = refs[:np_], refs[np_:2 * np_], refs[-1]
        t = jnp.dot(part_refs[0][...], w_refs[0][...],
                    preferred_element_type=_F32)
        for j in range(1, np_):
            t += jnp.dot(part_refs[j][...], w_refs[j][...],
                         preferred_element_type=_F32)
        o_ref[...] = t

    return pl.pallas_call(
        body,
        grid=(GRID,),
        in_specs=[_row_spec(HID)] * np_ + [_full_spec((HID, HID))] * np_,
        out_specs=_row_spec(HID),
        out_shape=jax.ShapeDtypeStruct((N, HID), _F32),
    )(*parts, *Ws)


def _post_mm(shs, deg, hb, eW, eb, eg, ebt, ypart, hW_last):
    def body(*refs):
        i = 0
        s_refs = refs[i:i + 2]; i += 2
        deg_ref, hb_ref = refs[i:i + 2]; i += 2
        ew_refs = refs[i:i + 2]; i += 2
        eb_ref, g_ref, bt_ref = refs[i:i + 3]; i += 3
        yp_ref, hw_ref = refs[i:i + 2]; i += 2
        h_ref = refs[i]; i += 1
        oh = refs[i:]

        h = _post_val([r[...] for r in s_refs], deg_ref, hb_ref, ew_refs,
                      eb_ref, g_ref, bt_ref)
        h_ref[...] = h
        y = yp_ref[...] + jnp.dot(h, hw_ref[...], preferred_element_type=_F32)
        _halves(y, oh)

    return pl.pallas_call(
        body,
        grid=(GRID,),
        in_specs=_POST_SPECS + [_row_spec(HID), _full_spec((HID, HID))],
        out_specs=[_row_spec(HID)] + [_row_spec(128)] * 2,
        out_shape=[jax.ShapeDtypeStruct((N, HID), _F32)]
                  + [jax.ShapeDtypeStruct((N, 128), _F32)] * 2,
    )(*shs, deg, hb, eW[:128], eW[128:], eb, eg, ebt, ypart, hW_last)


def _post_head(shs, deg, hb, eW, eb, eg, ebt, tpart, W3_last,
               b3, g3, bt3, fcW_pad, fcb_pad):
    def body(*refs):
        i = 0
        s_refs = refs[i:i + 2]; i += 2
        deg_ref, hb_ref = refs[i:i + 2]; i += 2
        ew_refs = refs[i:i + 2]; i += 2
        eb_ref, g_ref, bt_ref = refs[i:i + 3]; i += 3
        (tp_ref, w3_ref, b3_ref, g3_ref, bt3_ref,
         fw_ref, fb_ref, o_ref) = refs[i:]

        h = _post_val([r[...] for r in s_refs], deg_ref, hb_ref, ew_refs,
                      eb_ref, g_ref, bt_ref)
        t = tp_ref[...] + jnp.dot(h, w3_ref[...], preferred_element_type=_F32)
        z = _leaky_val(_ln_val(t + b3_ref[...], g3_ref[...], bt3_ref[...]))
        o_ref[...] = jnp.dot(z, fw_ref[...], preferred_element_type=_F32) \
            + fb_ref[...]

    return pl.pallas_call(
        body,
        grid=(GRID,),
        in_specs=_POST_SPECS
                 + [_row_spec(HID), _full_spec((HID, HID))]
                 + [_full_spec((1, HID))] * 3
                 + [_full_spec((HID, 128)), _full_spec((1, 128))],
        out_specs=_row_spec(128),
        out_shape=jax.ShapeDtypeStruct((N, 128), _F32),
    )(*shs, deg, hb, eW[:128], eW[128:], eb, eg, ebt, tpart, W3_last,
      b3, g3, bt3, fcW_pad, fcb_pad)


def kernel(x_all, edge_index, p, fc1_W, fc1_b, ln1_g, ln1_b,
           hW0, hb0, eW0, eb0, eg0, ebt0,
           hW1, hb1, eW1, eb1, eg1, ebt1,
           hW2, hb2, eW2, eb2, eg2, ebt2,
           fc3_W, fc3_b, ln3_g, ln3_b, fc_W, fc_b):
    del p
    hWs = [hW0, hW1, hW2]
    hbs = [hb0, hb1, hb2]
    eWs = [eW0, eW1, eW2]
    ebs = [eb0, eb1, eb2]
    egs = [eg0, eg1, eg2]
    ebts = [ebt0, ebt1, ebt2]

    row = lambda v: v.reshape(1, -1)

    npad = EP - E
    ar = jnp.arange(npad, dtype=jnp.int32)
    src = jnp.concatenate([edge_index[0], (ar * 37) % N]).reshape(NS, NCHUNK, CHUNK)
    dst = jnp.concatenate([edge_index[1], N + (ar % 16)]).reshape(NS, NCHUNK, CHUNK)

    deg = _deg(dst)
    x0, *yhs = _mm_ln_split(x_all, fc1_W, row(fc1_b), row(ln1_g), row(ln1_b),
                            hW0)

    def wsplit(W, n):
        return [lax.slice(W, (j * HID, 0), ((j + 1) * HID, HID))
                for j in range(n)]

    parts = [x0]
    for i in range(LAYERS - 1):
        shs = _seg_sum(*yhs, src, dst)
        hWn = wsplit(hWs[i + 1], i + 2)
        ypart = _mm_parts(parts, hWn[:i + 1])
        h, *yhs = _post_mm(shs, deg, row(hbs[i]), eWs[i], row(ebs[i]),
                           row(egs[i]), row(ebts[i]), ypart, hWn[i + 1])
        parts.append(h)

    shs = _seg_sum(*yhs, src, dst)
    W3s = wsplit(fc3_W, LAYERS + 1)
    tpart = _mm_parts(parts, W3s[:LAYERS])
    fcW_pad = jnp.pad(fc_W, ((0, 0), (0, 128 - NUM_CLASS)))
    fcb_pad = jnp.pad(fc_b, ((0, 128 - NUM_CLASS),))
    out = _post_head(shs, deg, row(hbs[2]), eWs[2], row(ebs[2]),
                     row(egs[2]), row(ebts[2]), tpart, W3s[LAYERS],
                     row(fc3_b), row(ln3_g), row(ln3_b),
                     fcW_pad, row(fcb_pad))
    return out[:, :NUM_CLASS]
